# baseline (speedup 1.0000x reference)
import functools

import jax
import jax.numpy as jnp
from jax import lax
from jax.experimental import pallas as pl
from jax.experimental.pallas import tpu as pltpu
from jax.experimental.pallas import tpu_sc as plsc

N = 10000
NPAD = 10240
ROWS_PER_TILE = NPAD // 16
NDUMP = 32
NC, NS = 2, 16
NW = NC * NS
CHUNK = 128


def _wid(c, s):
    return c * NS + s


def _deg_body(nchunks, dst2_hbm, out_hbm, acc, dall, ones_v, zb, dsem0, dsem1):
    c = lax.axis_index("c")
    s = lax.axis_index("s")
    w = _wid(c, s)

    def zero_body(k, _):
        zb[pl.ds(k * 16, 16)] = jnp.zeros((16,), jnp.float32)
        return 0

    lax.fori_loop(0, ROWS_PER_TILE // 16, zero_body, 0)
    for k in range(CHUNK // 16):
        ones_v[pl.ds(k * 16, 16)] = jnp.ones((16,), jnp.float32)
    pltpu.sync_copy(zb, acc.at[pl.ds(s * ROWS_PER_TILE, ROWS_PER_TILE)])
    pltpu.sync_copy(dst2_hbm.at[pl.ds(w * nchunks, nchunks)], dall)
    plsc.subcore_barrier()

    ngrp = 16
    dsem = (dsem0, dsem1)
    descs = [None, None]
    for g0 in range(0, nchunks, ngrp):
        par = (g0 // ngrp) % 2
        if descs[par] is not None:
            for dd in descs[par]:
                dd.wait()
        descs[par] = [
            pltpu.async_copy(ones_v, acc.at[dall.at[j]], dsem[par], add=True)
            for j in range(g0, min(g0 + ngrp, nchunks))]
    for par in range(2):
        if descs[par] is not None:
            for dd in descs[par]:
                dd.wait()
    plsc.subcore_barrier()
    pltpu.sync_copy(acc.at[pl.ds(s * ROWS_PER_TILE, ROWS_PER_TILE)],
                    out_hbm.at[c, pl.ds(s * ROWS_PER_TILE, ROWS_PER_TILE)])


def _deg_call(dst2, nchunks):
    mesh = plsc.VectorSubcoreMesh(core_axis_name="c", subcore_axis_name="s",
                                  num_cores=NC, num_subcores=NS)
    return pl.kernel(
        functools.partial(_deg_body, nchunks),
        out_type=jax.ShapeDtypeStruct((NC, NPAD), jnp.float32),
        mesh=mesh,
        scratch_types=[
            pltpu.VMEM_SHARED((NPAD,), jnp.float32),
            pltpu.VMEM((nchunks, CHUNK), jnp.int32),
            pltpu.VMEM((CHUNK,), jnp.float32),
            pltpu.VMEM((ROWS_PER_TILE,), jnp.float32),
            pltpu.SemaphoreType.DMA,
            pltpu.SemaphoreType.DMA,
        ],
    )(dst2)


NB = 16
NBC = NB * CHUNK


def _agg_body(nchunks, d, hs_hbm, src_hbm, dst2_hbm, out_hbm, acc,
              sblk0, sblk1, dblk0, dblk1, msg0, msg1,
              gsem0, gsem1, isem0, isem1, ssem0, ssem1):
    c = lax.axis_index("c")
    s = lax.axis_index("s")
    w = _wid(c, s)
    nblocks = nchunks // NB

    def zero_body(i, _):
        for k in range(d // 16):
            msg0[i, pl.ds(k * 16, 16)] = jnp.zeros((16,), jnp.float32)
        return 0

    lax.fori_loop(0, CHUNK, zero_body, 0)
    for k in range(ROWS_PER_TILE // CHUNK):
        pltpu.sync_copy(msg0, acc.at[pl.ds(s * ROWS_PER_TILE + k * CHUNK, CHUNK)])
    plsc.subcore_barrier()

    base = w * nchunks * CHUNK
    sblk = (sblk0, sblk1)
    dblk = (dblk0, dblk1)
    msg = (msg0, msg1)
    gsem = (gsem0, gsem1)
    ssem = (ssem0, ssem1)
    gdesc = [None, None]
    sdesc = [None, None]
    idesc = [None, None]

    pltpu.sync_copy(src_hbm.at[pl.ds(base, NBC)], sblk[0])
    pltpu.sync_copy(dst2_hbm.at[pl.ds(w * nchunks, NB)], dblk[0])
    gdesc[0] = pltpu.async_copy(hs_hbm.at[sblk[0].at[pl.ds(0, CHUNK)]],
                                msg[0], gsem[0])
    for b in range(nblocks):
        pb = b % 2
        nb_ = (b + 1) % 2
        if b + 1 < nblocks:
            idesc[0] = pltpu.async_copy(
                src_hbm.at[pl.ds(base + (b + 1) * NBC, NBC)], sblk[nb_], isem0)
            idesc[1] = pltpu.async_copy(
                dst2_hbm.at[pl.ds(w * nchunks + (b + 1) * NB, NB)],
                dblk[nb_], isem1)
        for j in range(NB):
            g = b * NB + j
            p = g % 2
            q = (g + 1) % 2
            if j + 1 < NB or b + 1 < nblocks:
                if sdesc[q] is not None:
                    sdesc[q].wait()
                    sdesc[q] = None
                if j + 1 < NB:
                    gidx = sblk[pb].at[pl.ds((j + 1) * CHUNK, CHUNK)]
                else:
                    idesc[0].wait()
                    idesc[1].wait()
                    gidx = sblk[nb_].at[pl.ds(0, CHUNK)]
                gdesc[q] = pltpu.async_copy(hs_hbm.at[gidx], msg[q], gsem[q])
            gdesc[p].wait()
            sdesc[p] = pltpu.async_copy(msg[p], acc.at[dblk[pb].at[j]],
                                        ssem[p], add=True)
    for p in range(2):
        if sdesc[p] is not None:
            sdesc[p].wait()
    plsc.subcore_barrier()
    odesc = []
    for k in range(ROWS_PER_TILE // CHUNK):
        r = s * ROWS_PER_TILE + k * CHUNK
        odesc.append(pltpu.async_copy(acc.at[pl.ds(r, CHUNK)],
                                      out_hbm.at[c, pl.ds(r, CHUNK)], gsem0))
    for dd in odesc:
        dd.wait()


def _agg_call(hs, src, dst2, nchunks, d):
    mesh = plsc.VectorSubcoreMesh(core_axis_name="c", subcore_axis_name="s",
                                  num_cores=NC, num_subcores=NS)
    return pl.kernel(
        functools.partial(_agg_body, nchunks, d),
        out_type=jax.ShapeDtypeStruct((NC, NPAD, d), jnp.float32),
        mesh=mesh,
        scratch_types=[
            pltpu.VMEM_SHARED((NPAD, d), jnp.float32),
            pltpu.VMEM((NBC,), jnp.int32),
            pltpu.VMEM((NBC,), jnp.int32),
            pltpu.VMEM((NB, CHUNK), jnp.int32),
            pltpu.VMEM((NB, CHUNK), jnp.int32),
            pltpu.VMEM((CHUNK, d), jnp.float32),
            pltpu.VMEM((CHUNK, d), jnp.float32),
            pltpu.SemaphoreType.DMA,
            pltpu.SemaphoreType.DMA,
            pltpu.SemaphoreType.DMA,
            pltpu.SemaphoreType.DMA,
            pltpu.SemaphoreType.DMA,
            pltpu.SemaphoreType.DMA,
        ],
    )(hs, src, dst2)


def _t1_body(x_ref, w_ref, degp_ref, hs_ref, dinv_ref):
    deg = degp_ref[0] + degp_ref[1] + 1.0
    dinv = lax.rsqrt(deg)
    dinv_ref[...] = dinv
    xw = jnp.dot(x_ref[...], w_ref[...], preferred_element_type=jnp.float32)
    hs_ref[:N] = dinv[:N] * xw
    hs_ref[N:] = jnp.zeros((NPAD - N, xw.shape[1]), jnp.float32)


def _t1_call(x, w1, degp):
    return pl.pallas_call(
        _t1_body,
        out_shape=[
            jax.ShapeDtypeStruct((NPAD, w1.shape[1]), jnp.float32),
            jax.ShapeDtypeStruct((NPAD, 1), jnp.float32),
        ],
    )(x, w1, degp)


def _t2_body(p_ref, hs_ref, dinv_ref, b_ref, g_ref, be_ref, w_ref, out_ref):
    dinv = dinv_ref[...]
    tot = p_ref[0] + p_ref[1] + hs_ref[...]
    agg = dinv * tot + b_ref[...]
    a = agg[:N]
    mu = jnp.mean(a, axis=0, keepdims=True)
    var = jnp.mean(a * a, axis=0, keepdims=True) - mu * mu
    h = (agg - mu) * lax.rsqrt(var + 1e-5) * g_ref[...] + be_ref[...]
    h = jnp.maximum(h, 0.0)
    out_ref[...] = dinv * jnp.dot(h, w_ref[...],
                                  preferred_element_type=jnp.float32)


def _t2_call(p, hs, dinv, b, g, be, w):
    return pl.pallas_call(
        _t2_body,
        out_shape=jax.ShapeDtypeStruct((NPAD, w.shape[1]), jnp.float32),
    )(p, hs, dinv, b, g, be, w)


def _t3_body(dout, p_ref, hs_ref, dinv_ref, b_ref, out_ref):
    z = dinv_ref[...] * (p_ref[0] + p_ref[1] + hs_ref[...]) + b_ref[...]
    z = z[:N, :dout]
    m = jnp.max(z, axis=1, keepdims=True)
    lse = jnp.log(jnp.sum(jnp.exp(z - m), axis=1, keepdims=True))
    out_ref[...] = z - m - lse


def _t3_call(p, hs, dinv, b, dout):
    return pl.pallas_call(
        functools.partial(_t3_body, dout),
        out_shape=jax.ShapeDtypeStruct((N, dout), jnp.float32),
    )(p, hs, dinv, b)


def kernel(x, edge_index, W1, b1, g1, be1, W2, b2, g2, be2, W3, b3):
    e = edge_index.shape[1]
    nchunks = -(-e // (NW * CHUNK))
    nchunks = -(-nchunks // NB) * NB
    epad = NW * CHUNK * nchunks
    padn = epad - e
    pad_ids = N + (jnp.arange(padn, dtype=jnp.int32) % NDUMP)
    src = jnp.concatenate([edge_index[0], pad_ids])
    dst = jnp.concatenate([edge_index[1], pad_ids])
    dst2 = jnp.reshape(dst, (epad // CHUNK, CHUNK))

    degp = _deg_call(dst2, nchunks)
    degp3 = jnp.reshape(degp, (NC, NPAD, 1))
    hs1, dinv = _t1_call(x, W1, degp3)

    p1 = _agg_call(hs1, src, dst2, nchunks, W1.shape[1])
    hs2 = _t2_call(p1, hs1, dinv, b1.reshape(1, -1), g1.reshape(1, -1),
                   be1.reshape(1, -1), W2)

    dout = W3.shape[1]
    w3_pad = jnp.concatenate(
        [W3, jnp.zeros((W3.shape[0], 128 - dout), jnp.float32)], axis=1)
    b3_pad = jnp.concatenate(
        [b3, jnp.zeros((128 - dout,), jnp.float32)]).reshape(1, -1)

    p2 = _agg_call(hs2, src, dst2, nchunks, W2.shape[1])
    hs3 = _t2_call(p2, hs2, dinv, b2.reshape(1, -1), g2.reshape(1, -1),
                   be2.reshape(1, -1), w3_pad)

    p3 = _agg_call(hs3, src, dst2, nchunks, 128)
    return _t3_call(p3, hs3, dinv, b3_pad, dout)


# speedup vs baseline: 1.0067x; 1.0067x over previous
import functools

import jax
import jax.numpy as jnp
from jax import lax
from jax.experimental import pallas as pl
from jax.experimental.pallas import tpu as pltpu
from jax.experimental.pallas import tpu_sc as plsc

N = 10000
NPAD = 10240
ROWS_PER_TILE = NPAD // 16
NDUMP = 32
NC, NS = 2, 16
NW = NC * NS
CHUNK = 128


def _wid(c, s):
    return c * NS + s


def _deg_body(nchunks, dst2_hbm, out_hbm, acc, dall, ones_v, zb, dsem0, dsem1):
    c = lax.axis_index("c")
    s = lax.axis_index("s")
    w = _wid(c, s)

    def zero_body(k, _):
        zb[pl.ds(k * 16, 16)] = jnp.zeros((16,), jnp.float32)
        return 0

    lax.fori_loop(0, ROWS_PER_TILE // 16, zero_body, 0)
    for k in range(CHUNK // 16):
        ones_v[pl.ds(k * 16, 16)] = jnp.ones((16,), jnp.float32)
    pltpu.sync_copy(zb, acc.at[pl.ds(s * ROWS_PER_TILE, ROWS_PER_TILE)])
    pltpu.sync_copy(dst2_hbm.at[pl.ds(w * nchunks, nchunks)], dall)
    plsc.subcore_barrier()

    ngrp = 16
    dsem = (dsem0, dsem1)
    descs = [None, None]
    for g0 in range(0, nchunks, ngrp):
        par = (g0 // ngrp) % 2
        if descs[par] is not None:
            for dd in descs[par]:
                dd.wait()
        descs[par] = [
            pltpu.async_copy(ones_v, acc.at[dall.at[j]], dsem[par], add=True)
            for j in range(g0, min(g0 + ngrp, nchunks))]
    for par in range(2):
        if descs[par] is not None:
            for dd in descs[par]:
                dd.wait()
    plsc.subcore_barrier()
    pltpu.sync_copy(acc.at[pl.ds(s * ROWS_PER_TILE, ROWS_PER_TILE)],
                    out_hbm.at[c, pl.ds(s * ROWS_PER_TILE, ROWS_PER_TILE)])


def _deg_call(dst2, nchunks):
    mesh = plsc.VectorSubcoreMesh(core_axis_name="c", subcore_axis_name="s",
                                  num_cores=NC, num_subcores=NS)
    return pl.kernel(
        functools.partial(_deg_body, nchunks),
        out_type=jax.ShapeDtypeStruct((NC, NPAD), jnp.float32),
        mesh=mesh,
        scratch_types=[
            pltpu.VMEM_SHARED((NPAD,), jnp.float32),
            pltpu.VMEM((nchunks, CHUNK), jnp.int32),
            pltpu.VMEM((CHUNK,), jnp.float32),
            pltpu.VMEM((ROWS_PER_TILE,), jnp.float32),
            pltpu.SemaphoreType.DMA,
            pltpu.SemaphoreType.DMA,
        ],
    )(dst2)


NB = 16
NBC = NB * CHUNK


def _agg_body(nchunks, d, hs_hbm, src_hbm, dst2_hbm, out_hbm, acc,
              sblk0, sblk1, dblk0, dblk1, msg0, msg1,
              gsem0, gsem1, isem0, isem1, ssem0, ssem1):
    c = lax.axis_index("c")
    s = lax.axis_index("s")
    w = _wid(c, s)
    nblocks = nchunks // NB

    def zero_body(i, _):
        for k in range(d // 16):
            msg0[i, pl.ds(k * 16, 16)] = jnp.zeros((16,), jnp.float32)
        return 0

    lax.fori_loop(0, CHUNK, zero_body, 0)
    for k in range(ROWS_PER_TILE // CHUNK):
        pltpu.sync_copy(msg0, acc.at[pl.ds(s * ROWS_PER_TILE + k * CHUNK, CHUNK)])
    plsc.subcore_barrier()

    base = w * nchunks * CHUNK
    sblk = (sblk0, sblk1)
    dblk = (dblk0, dblk1)
    msg = (msg0, msg1)
    gsem = (gsem0, gsem1)
    ssem = (ssem0, ssem1)
    gdesc = [None, None]
    sdesc = [None, None]
    idesc = [None, None]

    pltpu.sync_copy(src_hbm.at[pl.ds(base, NBC)], sblk[0])
    pltpu.sync_copy(dst2_hbm.at[pl.ds(w * nchunks, NB)], dblk[0])
    gdesc[0] = pltpu.async_copy(hs_hbm.at[sblk[0].at[pl.ds(0, CHUNK)]],
                                msg[0], gsem[0])
    for b in range(nblocks):
        pb = b % 2
        nb_ = (b + 1) % 2
        if b + 1 < nblocks:
            idesc[0] = pltpu.async_copy(
                src_hbm.at[pl.ds(base + (b + 1) * NBC, NBC)], sblk[nb_], isem0)
            idesc[1] = pltpu.async_copy(
                dst2_hbm.at[pl.ds(w * nchunks + (b + 1) * NB, NB)],
                dblk[nb_], isem1)
        for j in range(NB):
            g = b * NB + j
            p = g % 2
            q = (g + 1) % 2
            if j + 1 < NB or b + 1 < nblocks:
                if sdesc[q] is not None:
                    sdesc[q].wait()
                    sdesc[q] = None
                if j + 1 < NB:
                    gidx = sblk[pb].at[pl.ds((j + 1) * CHUNK, CHUNK)]
                else:
                    idesc[0].wait()
                    idesc[1].wait()
                    gidx = sblk[nb_].at[pl.ds(0, CHUNK)]
                gdesc[q] = pltpu.async_copy(hs_hbm.at[gidx], msg[q], gsem[q])
            gdesc[p].wait()
            sdesc[p] = pltpu.async_copy(msg[p], acc.at[dblk[pb].at[j]],
                                        ssem[p], add=True)
    for p in range(2):
        if sdesc[p] is not None:
            sdesc[p].wait()
    plsc.subcore_barrier()
    for k in range(ROWS_PER_TILE // CHUNK):
        r = s * ROWS_PER_TILE + k * CHUNK
        pltpu.sync_copy(acc.at[pl.ds(r, CHUNK)], out_hbm.at[c, pl.ds(r, CHUNK)])


def _agg_call(hs, src, dst2, nchunks, d):
    mesh = plsc.VectorSubcoreMesh(core_axis_name="c", subcore_axis_name="s",
                                  num_cores=NC, num_subcores=NS)
    return pl.kernel(
        functools.partial(_agg_body, nchunks, d),
        out_type=jax.ShapeDtypeStruct((NC, NPAD, d), jnp.float32),
        mesh=mesh,
        scratch_types=[
            pltpu.VMEM_SHARED((NPAD, d), jnp.float32),
            pltpu.VMEM((NBC,), jnp.int32),
            pltpu.VMEM((NBC,), jnp.int32),
            pltpu.VMEM((NB, CHUNK), jnp.int32),
            pltpu.VMEM((NB, CHUNK), jnp.int32),
            pltpu.VMEM((CHUNK, d), jnp.float32),
            pltpu.VMEM((CHUNK, d), jnp.float32),
            pltpu.SemaphoreType.DMA,
            pltpu.SemaphoreType.DMA,
            pltpu.SemaphoreType.DMA,
            pltpu.SemaphoreType.DMA,
            pltpu.SemaphoreType.DMA,
            pltpu.SemaphoreType.DMA,
        ],
    )(hs, src, dst2)


def _t1_body(x_ref, w_ref, degp_ref, hs_ref, dinv_ref):
    deg = degp_ref[0] + degp_ref[1] + 1.0
    dinv = lax.rsqrt(deg)
    dinv_ref[...] = dinv
    xw = jnp.dot(x_ref[...], w_ref[...], preferred_element_type=jnp.float32)
    hs_ref[:N] = dinv[:N] * xw
    hs_ref[N:] = jnp.zeros((NPAD - N, xw.shape[1]), jnp.float32)


def _t1_call(x, w1, degp):
    return pl.pallas_call(
        _t1_body,
        out_shape=[
            jax.ShapeDtypeStruct((NPAD, w1.shape[1]), jnp.float32),
            jax.ShapeDtypeStruct((NPAD, 1), jnp.float32),
        ],
    )(x, w1, degp)


def _t2_body(p_ref, hs_ref, dinv_ref, b_ref, g_ref, be_ref, w_ref, out_ref):
    dinv = dinv_ref[...]
    tot = p_ref[0] + p_ref[1] + hs_ref[...]
    agg = dinv * tot + b_ref[...]
    a = agg[:N]
    mu = jnp.mean(a, axis=0, keepdims=True)
    var = jnp.mean(a * a, axis=0, keepdims=True) - mu * mu
    h = (agg - mu) * lax.rsqrt(var + 1e-5) * g_ref[...] + be_ref[...]
    h = jnp.maximum(h, 0.0)
    out_ref[...] = dinv * jnp.dot(h, w_ref[...],
                                  preferred_element_type=jnp.float32)


def _t2_call(p, hs, dinv, b, g, be, w):
    return pl.pallas_call(
        _t2_body,
        out_shape=jax.ShapeDtypeStruct((NPAD, w.shape[1]), jnp.float32),
    )(p, hs, dinv, b, g, be, w)


def _t3_body(dout, p_ref, hs_ref, dinv_ref, b_ref, out_ref):
    z = dinv_ref[...] * (p_ref[0] + p_ref[1] + hs_ref[...]) + b_ref[...]
    z = z[:N, :dout]
    m = jnp.max(z, axis=1, keepdims=True)
    lse = jnp.log(jnp.sum(jnp.exp(z - m), axis=1, keepdims=True))
    out_ref[...] = z - m - lse


def _t3_call(p, hs, dinv, b, dout):
    return pl.pallas_call(
        functools.partial(_t3_body, dout),
        out_shape=jax.ShapeDtypeStruct((N, dout), jnp.float32),
    )(p, hs, dinv, b)


def kernel(x, edge_index, W1, b1, g1, be1, W2, b2, g2, be2, W3, b3):
    e = edge_index.shape[1]
    nchunks = -(-e // (NW * CHUNK))
    nchunks = -(-nchunks // NB) * NB
    epad = NW * CHUNK * nchunks
    padn = epad - e
    pad_ids = N + (jnp.arange(padn, dtype=jnp.int32) % NDUMP)
    src = jnp.concatenate([edge_index[0], pad_ids])
    dst = jnp.concatenate([edge_index[1], pad_ids])
    dst2 = jnp.reshape(dst, (epad // CHUNK, CHUNK))

    degp = _deg_call(dst2, nchunks)
    degp3 = jnp.reshape(degp, (NC, NPAD, 1))
    hs1, dinv = _t1_call(x, W1, degp3)

    p1 = _agg_call(hs1, src, dst2, nchunks, W1.shape[1])
    hs2 = _t2_call(p1, hs1, dinv, b1.reshape(1, -1), g1.reshape(1, -1),
                   be1.reshape(1, -1), W2)

    dout = W3.shape[1]
    w3_pad = jnp.concatenate(
        [W3, jnp.zeros((W3.shape[0], 128 - dout), jnp.float32)], axis=1)
    b3_pad = jnp.concatenate(
        [b3, jnp.zeros((128 - dout,), jnp.float32)]).reshape(1, -1)

    p2 = _agg_call(hs2, src, dst2, nchunks, W2.shape[1])
    hs3 = _t2_call(p2, hs2, dinv, b2.reshape(1, -1), g2.reshape(1, -1),
                   be2.reshape(1, -1), w3_pad)

    p3 = _agg_call(hs3, src, dst2, nchunks, 128)
    return _t3_call(p3, hs3, dinv, b3_pad, dout)
